# baseline (speedup 1.0000x reference)
import jax
import jax.numpy as jnp
from jax.experimental import pallas as pl
from jax.experimental.pallas import tpu as pltpu

_DIM = 384
_HID = 1536
_TBL = 128


def _table_body(Wo_ref, bo_ref, g_ref, b_ref, W1_ref, b1_ref, W2_ref, b2_ref,
                ls_ref, tbl_ref):
    e = jax.lax.broadcasted_iota(jnp.int32, (_TBL, 1), 0)
    codes = jnp.concatenate(
        [((e // d) % 3 - 1) for d in (27, 9, 3, 1)],
        axis=1).astype(jnp.float32)
    zq = jnp.dot(codes, Wo_ref[...],
                 preferred_element_type=jnp.float32) + bo_ref[...]
    mu = jnp.mean(zq, axis=-1, keepdims=True)
    var = jnp.mean((zq - mu) ** 2, axis=-1, keepdims=True)
    h = (zq - mu) / jnp.sqrt(var + 1e-5) * g_ref[...] + b_ref[...]
    h1 = jnp.dot(h, W1_ref[...],
                 preferred_element_type=jnp.float32) + b1_ref[...]
    h1 = 0.5 * h1 * (1.0 + jax.lax.erf(h1 / jnp.sqrt(2.0).astype(jnp.float32)))
    h2 = jnp.dot(h1, W2_ref[...],
                 preferred_element_type=jnp.float32) + b2_ref[...]
    tbl_ref[...] = h2 * ls_ref[...]


def _stream_body(x_ref, Wi_ref, bi_ref, tbl_ref, T_ref,
                 out_ref, feat0_ref):
    xb = x_ref[...]
    z = jnp.dot(xb, Wi_ref[...],
                preferred_element_type=jnp.float32) + bi_ref[...]
    t = T_ref[0, 0]
    q = jnp.round(jnp.tanh(t * z) * 0.999)
    idx = (((q[:, 0:1] * 3.0 + q[:, 1:2]) * 3.0 + q[:, 2:3]) * 3.0
           + q[:, 3:4]).astype(jnp.int32) + 40
    onehot = (idx == jax.lax.broadcasted_iota(
        jnp.int32, (xb.shape[0], _TBL), 1)).astype(jnp.float32)
    res = xb + jnp.dot(onehot, tbl_ref[...],
                       preferred_element_type=jnp.float32)
    out_ref[...] = res
    feat0_ref[...] = xb


def _run(x2, Wi, bi, Wo, bo, g, b, W1, b1, W2, b2, ls, T):
    BN = x2.shape[0]
    R = 2048
    full = lambda shape: pl.BlockSpec(shape, lambda *_: (0,) * len(shape))
    table = pl.pallas_call(
        _table_body,
        in_specs=[
            full((4, _DIM)), full((1, _DIM)),
            full((1, _DIM)), full((1, _DIM)),
            full((_DIM, _HID)), full((1, _HID)),
            full((_HID, _DIM)), full((1, _DIM)),
            full((1, _DIM)),
        ],
        out_specs=full((_TBL, _DIM)),
        out_shape=jax.ShapeDtypeStruct((_TBL, _DIM), jnp.float32),
    )(Wo, bo, g, b, W1, b1, W2, b2, ls)
    return pl.pallas_call(
        _stream_body,
        grid=(BN // R,),
        in_specs=[
            pl.BlockSpec((R, _DIM), lambda i: (i, 0)),
            full((_DIM, 4)), full((1, 4)),
            full((_TBL, _DIM)),
            pl.BlockSpec(memory_space=pltpu.SMEM),
        ],
        out_specs=[pl.BlockSpec((R, _DIM), lambda i: (i, 0))] * 2,
        out_shape=[jax.ShapeDtypeStruct((BN, _DIM), jnp.float32)] * 2,
        compiler_params=pltpu.CompilerParams(
            dimension_semantics=("parallel",)),
    )(x2, Wi, bi, table, T)


def kernel(x, W_in, b_in, W_out, b_out, ln_g, ln_b, W1, b1, W2, b2, gamma_ls, T):
    B, N, D = x.shape
    x2 = x.reshape(B * N, D)
    out2, feat02 = _run(
        x2, W_in, b_in.reshape(1, -1), W_out, b_out.reshape(1, -1),
        ln_g.reshape(1, -1), ln_b.reshape(1, -1), W1, b1.reshape(1, -1),
        W2, b2.reshape(1, -1), gamma_ls.reshape(1, -1),
        T.reshape(1, 1).astype(jnp.float32))
    loss = jnp.asarray(0.0, dtype=jnp.float32)
    out = out2.reshape(B, N, D)
    return (out, loss, feat02.reshape(B, N, D), out)


# speedup vs baseline: 1.2682x; 1.2682x over previous
import jax
import jax.numpy as jnp
from jax.experimental import pallas as pl
from jax.experimental.pallas import tpu as pltpu

_DIM = 384
_HID = 1536
_TBL = 128


def _table_body(Wo_ref, bo_ref, g_ref, b_ref, W1_ref, b1_ref, W2_ref, b2_ref,
                ls_ref, tbl_ref):
    e = jax.lax.broadcasted_iota(jnp.int32, (_TBL, 1), 0)
    codes = jnp.concatenate(
        [((e // d) % 3 - 1) for d in (27, 9, 3, 1)],
        axis=1).astype(jnp.float32)
    zq = jnp.dot(codes, Wo_ref[...],
                 preferred_element_type=jnp.float32) + bo_ref[...]
    mu = jnp.mean(zq, axis=-1, keepdims=True)
    var = jnp.mean((zq - mu) ** 2, axis=-1, keepdims=True)
    h = (zq - mu) / jnp.sqrt(var + 1e-5) * g_ref[...] + b_ref[...]
    h1 = jnp.dot(h, W1_ref[...],
                 preferred_element_type=jnp.float32) + b1_ref[...]
    h1 = 0.5 * h1 * (1.0 + jax.lax.erf(h1 / jnp.sqrt(2.0).astype(jnp.float32)))
    h2 = jnp.dot(h1, W2_ref[...],
                 preferred_element_type=jnp.float32) + b2_ref[...]
    tbl_ref[...] = h2 * ls_ref[...]


def _stream_body(x_ref, Wi_ref, bi_ref, tbl_ref, T_ref,
                 out_ref, feat0_ref, feat_ref):
    xb = x_ref[...]
    z = jnp.dot(xb, Wi_ref[...],
                preferred_element_type=jnp.float32) + bi_ref[...]
    t = T_ref[0, 0]
    q = jnp.round(jnp.tanh(t * z) * 0.999)
    idx = (((q[:, 0:1] * 3.0 + q[:, 1:2]) * 3.0 + q[:, 2:3]) * 3.0
           + q[:, 3:4]).astype(jnp.int32) + 40
    onehot = (idx == jax.lax.broadcasted_iota(
        jnp.int32, (xb.shape[0], _TBL), 1)).astype(jnp.float32)
    res = xb + jnp.dot(onehot, tbl_ref[...],
                       preferred_element_type=jnp.float32)
    out_ref[...] = res
    feat0_ref[...] = xb
    feat_ref[...] = res


def _run(x2, Wi, bi, Wo, bo, g, b, W1, b1, W2, b2, ls, T):
    BN = x2.shape[0]
    R = 4096
    full = lambda shape: pl.BlockSpec(shape, lambda *_: (0,) * len(shape))
    table = pl.pallas_call(
        _table_body,
        in_specs=[
            full((4, _DIM)), full((1, _DIM)),
            full((1, _DIM)), full((1, _DIM)),
            full((_DIM, _HID)), full((1, _HID)),
            full((_HID, _DIM)), full((1, _DIM)),
            full((1, _DIM)),
        ],
        out_specs=full((_TBL, _DIM)),
        out_shape=jax.ShapeDtypeStruct((_TBL, _DIM), jnp.float32),
    )(Wo, bo, g, b, W1, b1, W2, b2, ls)
    return pl.pallas_call(
        _stream_body,
        grid=(BN // R,),
        in_specs=[
            pl.BlockSpec((R, _DIM), lambda i: (i, 0)),
            full((_DIM, 4)), full((1, 4)),
            full((_TBL, _DIM)),
            pl.BlockSpec(memory_space=pltpu.SMEM),
        ],
        out_specs=[pl.BlockSpec((R, _DIM), lambda i: (i, 0))] * 3,
        out_shape=[jax.ShapeDtypeStruct((BN, _DIM), jnp.float32)] * 3,
        compiler_params=pltpu.CompilerParams(
            dimension_semantics=("parallel",)),
    )(x2, Wi, bi, table, T)


def kernel(x, W_in, b_in, W_out, b_out, ln_g, ln_b, W1, b1, W2, b2, gamma_ls, T):
    B, N, D = x.shape
    x2 = x.reshape(B * N, D)
    out2, feat02, feat2 = _run(
        x2, W_in, b_in.reshape(1, -1), W_out, b_out.reshape(1, -1),
        ln_g.reshape(1, -1), ln_b.reshape(1, -1), W1, b1.reshape(1, -1),
        W2, b2.reshape(1, -1), gamma_ls.reshape(1, -1),
        T.reshape(1, 1).astype(jnp.float32))
    loss = jnp.asarray(0.0, dtype=jnp.float32)
    return (out2.reshape(B, N, D), loss,
            feat02.reshape(B, N, D), feat2.reshape(B, N, D))
